# Initial kernel scaffold; baseline (speedup 1.0000x reference)
import jax
import jax.numpy as jnp
from jax.experimental import pallas as pl


def kernel(seq, W, word_table):
    raise NotImplementedError("write your pallas kernel here")


# speedup vs baseline: 4.1185x; 4.1185x over previous
import functools

import jax
import jax.numpy as jnp
from jax import lax
from jax.experimental import pallas as pl
from jax.experimental.pallas import tpu as pltpu
from jax.experimental.pallas import tpu_sc as plsc

_VOCAB = 100000
_EMB = 64
_REGION = 5
_NW = 32
_CP = 98
_NVEC = _EMB // 16


def _sc_kernel(w_hbm, word_hbm, idx_hbm, out_hbm,
               idx_v, unit_v, word_v, out_v,
               isem0, isem1, gsem0, gsem1, ssem0, ssem1):
  nchunks_per_w = idx_hbm.shape[0] // _NW
  wid = lax.axis_index("c") * 16 + lax.axis_index("s")
  base = wid * nchunks_per_w
  isems = (isem0, isem1)
  gsems = (gsem0, gsem1)
  ssems = (ssem0, ssem1)

  def fire_idx(jj, buf):
    pltpu.async_copy(idx_hbm.at[base + jj], idx_v.at[buf], isems[buf])

  def fire_gathers(buf):
    for r in range(_REGION):
      pltpu.async_copy(
          w_hbm.at[idx_v.at[buf, r]],
          unit_v.at[buf, pl.ds(r * _CP, _CP)],
          gsems[buf])
    pltpu.async_copy(word_hbm.at[idx_v.at[buf, _REGION]],
                     word_v.at[buf], gsems[buf])

  def wait_gathers(buf):
    pltpu.make_async_copy(w_hbm.at[pl.ds(0, _REGION * _CP)],
                          unit_v.at[buf], gsems[buf]).wait()
    pltpu.make_async_copy(word_hbm.at[pl.ds(0, _CP)],
                          word_v.at[buf], gsems[buf]).wait()

  for buf in range(2):
    fire_idx(buf, buf)
  for buf in range(2):
    pltpu.make_async_copy(idx_hbm.at[base], idx_v.at[buf],
                          isems[buf]).wait()
    fire_gathers(buf)

  @pl.loop(0, nchunks_per_w, step=2)
  def _outer(j):
    for buf in range(2):
      jj = j + buf
      wait_gathers(buf)

      @pl.when(jj + 2 < nchunks_per_w)
      def _():
        fire_idx(jj + 2, buf)

      @pl.when(jj >= 2)
      def _():
        pltpu.make_async_copy(out_hbm.at[pl.ds(0, _CP)],
                              out_v.at[buf], ssems[buf]).wait()

      @pl.loop(0, _CP)
      def _pos(i):
        for k in range(_NVEC):
          sl = pl.ds(16 * k, 16)
          w = word_v[buf, i, sl]
          acc = unit_v[buf, i, sl] * w
          for r in range(1, _REGION):
            acc = jnp.maximum(acc, unit_v[buf, r * _CP + i, sl] * w)
          out_v[buf, i, sl] = acc

      pltpu.async_copy(out_v.at[buf],
                       out_hbm.at[pl.ds((base + jj) * _CP, _CP)],
                       ssems[buf])

      @pl.when(jj + 2 < nchunks_per_w)
      def _():
        pltpu.make_async_copy(idx_hbm.at[base], idx_v.at[buf],
                              isems[buf]).wait()
        fire_gathers(buf)

  for buf in range(2):
    pltpu.make_async_copy(out_hbm.at[pl.ds(0, _CP)],
                          out_v.at[buf], ssems[buf]).wait()


@jax.jit
def kernel(seq, W, word_table):
  B, L = seq.shape
  radius = _REGION // 2
  Lc = L - 2 * radius
  seq = seq.astype(jnp.int32)

  rows = [seq[:, r:r + Lc] + jnp.int32(r * _VOCAB) for r in range(_REGION)]
  rows.append(seq[:, radius:radius + Lc])
  idx = jnp.stack(rows, axis=1)
  nch_per_row = Lc // _CP
  idx = idx.reshape(B, _REGION + 1, nch_per_row, _CP)
  idx = idx.transpose(0, 2, 1, 3).reshape(B * nch_per_row, _REGION + 1, _CP)

  nchunks = B * nch_per_row
  mesh = plsc.VectorSubcoreMesh(core_axis_name="c", subcore_axis_name="s")
  run = pl.kernel(
      _sc_kernel,
      out_type=jax.ShapeDtypeStruct((nchunks * _CP, _EMB), jnp.float32),
      mesh=mesh,
      compiler_params=pltpu.CompilerParams(use_tc_tiling_on_sc=False),
      scratch_types=[
          pltpu.VMEM((2, _REGION + 1, _CP), jnp.int32),
          pltpu.VMEM((2, _REGION * _CP, _EMB), jnp.float32),
          pltpu.VMEM((2, _CP, _EMB), jnp.float32),
          pltpu.VMEM((2, _CP, _EMB), jnp.float32),
          pltpu.SemaphoreType.DMA,
          pltpu.SemaphoreType.DMA,
          pltpu.SemaphoreType.DMA,
          pltpu.SemaphoreType.DMA,
          pltpu.SemaphoreType.DMA,
          pltpu.SemaphoreType.DMA,
      ],
  )
  out = run(W, word_table, idx)
  return out.reshape(B, Lc, _EMB)
